# baseline (speedup 1.0000x reference)
import functools

import jax
import jax.numpy as jnp
from jax import lax
from jax.experimental import pallas as pl
from jax.experimental.pallas import tpu as pltpu
from jax.experimental.pallas import tpu_sc as plsc

B = 4096
EMB = 32
LANES = 128
_CH = 8


@functools.cache
def _make_sc_gather(n):
    info = plsc.get_sparse_core_info()
    nc, ns = info.num_cores, info.num_subcores
    nw = nc * ns
    b_per_w = n // nw
    rounds = b_per_w // 16
    aligned = b_per_w % LANES == 0
    out_type = (jax.ShapeDtypeStruct((EMB, n), jnp.float32) if aligned
                else jax.ShapeDtypeStruct((EMB * n,), jnp.float32))
    rows_shape = (EMB, b_per_w) if aligned else (EMB * b_per_w,)

    mesh = plsc.VectorSubcoreMesh(core_axis_name="c", subcore_axis_name="s")

    @functools.partial(
        pl.kernel,
        mesh=mesh,
        compiler_params=pltpu.CompilerParams(needs_layout_passes=False),
        out_type=out_type,
        scratch_types=[
            pltpu.VMEM((n,), jnp.int32),
            pltpu.VMEM((_CH, EMB, LANES), jnp.float32),
            pltpu.VMEM(rows_shape, jnp.float32),
            pltpu.SemaphoreType.DMA,
        ],
    )
    def sc_gather(idx_hbm, tabT_hbm, outT_hbm, idx_all, blk, rows, sem):
        wid = lax.axis_index("s") * nc + lax.axis_index("c")
        base = pl.multiple_of(wid * b_per_w, b_per_w)
        pltpu.sync_copy(idx_hbm, idx_all)

        row16 = lax.iota(jnp.int32, 16)

        def round_body(r, _):
            start = pl.multiple_of(base + r * 16, 16)
            vec = idx_all[pl.ds(start, 16)]
            for h in range(16 // _CH):
                copies = []
                for j in range(_CH):
                    u = vec[h * _CH + j]
                    off = pl.multiple_of((u // LANES) * LANES, LANES)
                    copies.append(pltpu.async_copy(
                        tabT_hbm.at[:, pl.ds(off, LANES)], blk.at[j], sem))
                for c in copies:
                    c.wait()
                for j in range(_CH):
                    col = r * 16 + h * _CH + j
                    j16 = jnp.full((16,), j, dtype=jnp.int32)
                    lane16 = jnp.full(
                        (16,), vec[h * _CH + j] % LANES, dtype=jnp.int32)
                    lo = plsc.load_gather(blk, [j16, row16, lane16])
                    hi = plsc.load_gather(blk, [j16, row16 + 16, lane16])
                    if aligned:
                        c16 = jnp.full((16,), col, dtype=jnp.int32)
                        plsc.store_scatter(rows, [row16, c16], lo)
                        plsc.store_scatter(rows, [row16 + 16, c16], hi)
                    else:
                        flat = row16 * b_per_w + col
                        plsc.store_scatter(rows, [flat], lo)
                        plsc.store_scatter(rows, [flat + 16 * b_per_w], hi)
            return ()

        lax.fori_loop(0, rounds, round_body, ())
        if aligned:
            pltpu.sync_copy(rows, outT_hbm.at[:, pl.ds(base, b_per_w)])
        else:
            fbase = pl.multiple_of(wid * (EMB * b_per_w), 8)
            pltpu.sync_copy(rows, outT_hbm.at[pl.ds(fbase, EMB * b_per_w)])

    def call(idx, tabT):
        raw = sc_gather(idx, tabT)
        if aligned:
            return raw
        return (raw.reshape(nw, EMB, b_per_w)
                .transpose(1, 0, 2).reshape(EMB, n))

    return call


_BM = 512
_HALF = B // 2


def _mm_first_body(a_ref, b_ref, o_ref):
    o_ref[...] = lax.dot_general(
        a_ref[...], b_ref[...],
        (((0,), (0,)), ((), ())),
        preferred_element_type=jnp.float32,
    )


def _mm_second_body(a_ref, b_ref, prev_ref, o_ref):
    del prev_ref
    o_ref[...] = lax.dot_general(
        a_ref[...], b_ref[...],
        (((0,), (0,)), ((), ())),
        preferred_element_type=jnp.float32,
    )


def _score_matmul(uT0, uT1, iT):
    nsteps = _HALF // _BM
    top = pl.pallas_call(
        _mm_first_body,
        grid=(nsteps,),
        in_specs=[
            pl.BlockSpec((EMB, _BM), lambda i: (0, i)),
            pl.BlockSpec((EMB, B), lambda i: (0, 0)),
        ],
        out_specs=pl.BlockSpec((_BM, B), lambda i: (i, 0)),
        out_shape=jax.ShapeDtypeStruct((B, B), jnp.float32),
    )(uT0, iT)
    return pl.pallas_call(
        _mm_second_body,
        grid=(nsteps,),
        in_specs=[
            pl.BlockSpec((EMB, _BM), lambda i: (0, i)),
            pl.BlockSpec((EMB, B), lambda i: (0, 0)),
            pl.BlockSpec(memory_space=pl.ANY),
        ],
        out_specs=pl.BlockSpec((_BM, B), lambda i: (i + nsteps, 0)),
        out_shape=jax.ShapeDtypeStruct((B, B), jnp.float32),
        input_output_aliases={2: 0},
    )(uT1, iT, top)


def kernel(users, pos_items, user_table, item_table):
    utT = user_table.T
    iT = _make_sc_gather(B)(pos_items, item_table.T)
    u0_idx, _ = lax.optimization_barrier((users[:_HALF], iT))
    uT0 = _make_sc_gather(_HALF)(u0_idx, utT)
    u1_idx, _ = lax.optimization_barrier((users[_HALF:], uT0))
    uT1 = _make_sc_gather(_HALF)(u1_idx, utT)
    return _score_matmul(uT0, uT1, iT)
